# baseline (speedup 1.0000x reference)
import functools

import jax
import jax.numpy as jnp
from jax import lax
from jax.experimental import pallas as pl
from jax.experimental.pallas import tpu as pltpu
from jax.experimental.pallas import tpu_sc as plsc

N, E, D, DG, L = 10000, 320000, 128, 64, 4
NPAD = 10240
NC, NS = 2, 16
NW = NC * NS
EPW = E // NW
K = 125
NCHUNK = EPW // K
NBUF = 4
NROUND = NCHUNK // NBUF
RPT = NPAD // NS
DW = 16

_PERM = sum(([32 * j + 2 * i for i in range(16)]
             + [32 * j + 2 * i + 1 for i in range(16)] for j in range(2)), [])

_mesh = plsc.VectorSubcoreMesh(
    core_axis_name="c", subcore_axis_name="s", num_cores=NC, num_subcores=NS)
_sc_params = pltpu.CompilerParams(use_tc_tiling_on_sc=False,
                                  needs_layout_passes=False)


def _bf16_expand(rows_bf, rows_f, b):
    himask = jnp.full((16,), -65536, jnp.int32)

    def _cv(r, carry):
        for jj in range(2):
            v = rows_bf[b, r, pl.ds(jj * 32, 32)]
            u = plsc.bitcast(v, jnp.int32)
            lo = plsc.bitcast(u << 16, jnp.float32)
            hi = plsc.bitcast(u & himask, jnp.float32)
            rows_f[b, r, pl.ds(jj * 32, 16)] = lo
            rows_f[b, r, pl.ds(jj * 32 + 16, 16)] = hi
        return carry

    lax.fori_loop(0, K, _cv, 0)


@functools.partial(
    pl.kernel,
    out_type=jax.ShapeDtypeStruct((NC, NPAD, DG), jnp.float32),
    mesh=_mesh,
    compiler_params=_sc_params,
    scratch_types=[
        pltpu.VMEM((NCHUNK, K), jnp.int32),
        pltpu.VMEM((NCHUNK, K), jnp.int32),
        pltpu.VMEM((NBUF, K, DG), jnp.bfloat16),
        pltpu.VMEM((NBUF, K, DG), jnp.float32),
        pltpu.VMEM((64, DG), jnp.float32),
        pltpu.VMEM_SHARED((NPAD, DG), jnp.float32),
        pltpu.SemaphoreType.DMA((NBUF,)),
        pltpu.SemaphoreType.DMA((NBUF,)),
        pltpu.SemaphoreType.DMA,
        pltpu.SemaphoreType.DMA,
    ],
)
def _sc_segsum(edge_hbm, z_hbm, out_hbm, sidx, didx, rows, rowsf, zbuf, acc,
               gsem, ssem, isem, zsem):
    c = lax.axis_index("c")
    s = lax.axis_index("s")
    wid = s * NC + c
    zero16 = jnp.zeros((16,), jnp.float32)

    ild0 = pltpu.async_copy(edge_hbm.at[0, pl.ds(wid * NCHUNK, NCHUNK)], sidx,
                            isem)
    ild1 = pltpu.async_copy(edge_hbm.at[1, pl.ds(wid * NCHUNK, NCHUNK)], didx,
                            isem)

    def _zb(i, carry):
        zbuf[i // (DG // 16), pl.ds((i % (DG // 16)) * 16, 16)] = zero16
        return carry

    lax.fori_loop(0, 64 * (DG // 16), _zb, 0)

    def _zc(j, carry):
        pltpu.async_copy(zbuf, acc.at[pl.ds(s * RPT + j * 64, 64)], zsem)
        return carry

    lax.fori_loop(0, RPT // 64, _zc, 0)

    def _zd(j, carry):
        pltpu.make_async_copy(zbuf, acc.at[pl.ds(0, 64)], zsem).wait()
        return carry

    lax.fori_loop(0, RPT // 64, _zd, 0)
    ild0.wait()
    ild1.wait()
    plsc.subcore_barrier()

    for b in range(NBUF):
        pltpu.async_copy(z_hbm.at[sidx.at[b]], rows.at[b], gsem.at[b])

    def _round(g, carry):
        for b in range(NBUF):
            pltpu.make_async_copy(z_hbm.at[pl.ds(0, K)], rows.at[b],
                                  gsem.at[b]).wait()

            @pl.when(g > 0)
            def _():
                pltpu.make_async_copy(rowsf.at[b], acc.at[didx.at[0]],
                                      ssem.at[b]).wait()

            _bf16_expand(rows, rowsf, b)
            pltpu.async_copy(rowsf.at[b], acc.at[didx.at[g * NBUF + b]],
                             ssem.at[b], add=True)

            @pl.when(g + 1 < NROUND)
            def _():
                pltpu.async_copy(z_hbm.at[sidx.at[(g + 1) * NBUF + b]],
                                 rows.at[b], gsem.at[b])
        return carry

    lax.fori_loop(0, NROUND, _round, 0)
    for b in range(NBUF):
        pltpu.make_async_copy(rowsf.at[b], acc.at[didx.at[0]],
                              ssem.at[b]).wait()
    plsc.subcore_barrier()
    pltpu.sync_copy(acc.at[pl.ds(s * RPT, RPT)],
                    out_hbm.at[c, pl.ds(s * RPT, RPT)])


@functools.partial(
    pl.kernel,
    out_type=(jax.ShapeDtypeStruct((NC, NPAD, DG), jnp.float32),
              jax.ShapeDtypeStruct((NC, NPAD, DW), jnp.float32)),
    mesh=_mesh,
    compiler_params=_sc_params,
    scratch_types=[
        pltpu.VMEM((NCHUNK, K), jnp.int32),
        pltpu.VMEM((NCHUNK, K), jnp.int32),
        pltpu.VMEM((NBUF, K, DG), jnp.bfloat16),
        pltpu.VMEM((NBUF, K, DG), jnp.float32),
        pltpu.VMEM((64, DG), jnp.float32),
        pltpu.VMEM((K, DW), jnp.float32),
        pltpu.VMEM((64, DW), jnp.float32),
        pltpu.VMEM_SHARED((NPAD, DG), jnp.float32),
        pltpu.VMEM_SHARED((NPAD, DW), jnp.float32),
        pltpu.SemaphoreType.DMA((NBUF,)),
        pltpu.SemaphoreType.DMA((NBUF,)),
        pltpu.SemaphoreType.DMA,
        pltpu.SemaphoreType.DMA,
        pltpu.SemaphoreType.DMA,
    ],
)
def _sc_segsum_deg(edge_hbm, z_hbm, out_hbm, deg_hbm, sidx, didx, rows, rowsf,
                   zbuf, ones, zbuf2, acc, dacc, gsem, ssem, isem, zsem, dsem):
    c = lax.axis_index("c")
    s = lax.axis_index("s")
    wid = s * NC + c
    zero16 = jnp.zeros((16,), jnp.float32)
    one16 = jnp.ones((16,), jnp.float32)

    ild0 = pltpu.async_copy(edge_hbm.at[0, pl.ds(wid * NCHUNK, NCHUNK)], sidx,
                            isem)
    ild1 = pltpu.async_copy(edge_hbm.at[1, pl.ds(wid * NCHUNK, NCHUNK)], didx,
                            isem)

    def _zb(i, carry):
        zbuf[i // (DG // 16), pl.ds((i % (DG // 16)) * 16, 16)] = zero16
        return carry

    lax.fori_loop(0, 64 * (DG // 16), _zb, 0)

    def _fill(i, carry):
        ones[i, :] = one16
        zbuf2[i % 64, :] = zero16
        return carry

    lax.fori_loop(0, K, _fill, 0)

    def _zc(j, carry):
        pltpu.async_copy(zbuf, acc.at[pl.ds(s * RPT + j * 64, 64)], zsem)
        pltpu.async_copy(zbuf2, dacc.at[pl.ds(s * RPT + j * 64, 64)], zsem)
        return carry

    lax.fori_loop(0, RPT // 64, _zc, 0)

    def _zd(j, carry):
        pltpu.make_async_copy(zbuf, acc.at[pl.ds(0, 64)], zsem).wait()
        pltpu.make_async_copy(zbuf2, dacc.at[pl.ds(0, 64)], zsem).wait()
        return carry

    lax.fori_loop(0, RPT // 64, _zd, 0)
    ild0.wait()
    ild1.wait()
    plsc.subcore_barrier()

    for b in range(NBUF):
        pltpu.async_copy(z_hbm.at[sidx.at[b]], rows.at[b], gsem.at[b])

    def _round(g, carry):
        for b in range(NBUF):
            i = g * NBUF + b
            pltpu.make_async_copy(z_hbm.at[pl.ds(0, K)], rows.at[b],
                                  gsem.at[b]).wait()

            @pl.when(g > 0)
            def _():
                pltpu.make_async_copy(rowsf.at[b], acc.at[didx.at[0]],
                                      ssem.at[b]).wait()

            _bf16_expand(rows, rowsf, b)
            pltpu.async_copy(rowsf.at[b], acc.at[didx.at[i]],
                             ssem.at[b], add=True)
            pltpu.async_copy(ones, dacc.at[didx.at[i]], dsem, add=True)

            @pl.when(g + 1 < NROUND)
            def _():
                pltpu.async_copy(z_hbm.at[sidx.at[(g + 1) * NBUF + b]],
                                 rows.at[b], gsem.at[b])
        return carry

    lax.fori_loop(0, NROUND, _round, 0)
    for b in range(NBUF):
        pltpu.make_async_copy(rowsf.at[b], acc.at[didx.at[0]],
                              ssem.at[b]).wait()

    def _ddrain(i, carry):
        pltpu.make_async_copy(ones, dacc.at[didx.at[0]], dsem).wait()
        return carry

    lax.fori_loop(0, NCHUNK, _ddrain, 0)
    plsc.subcore_barrier()
    pltpu.sync_copy(acc.at[pl.ds(s * RPT, RPT)],
                    out_hbm.at[c, pl.ds(s * RPT, RPT)])
    pltpu.sync_copy(dacc.at[pl.ds(s * RPT, RPT)],
                    deg_hbm.at[c, pl.ds(s * RPT, RPT)])


def _ln(x, g, b, eps=1e-5):
    m = jnp.mean(x, axis=-1, keepdims=True)
    v = jnp.mean((x - m) ** 2, axis=-1, keepdims=True)
    return (x - m) / jnp.sqrt(v + eps) * g + b


_SQRT_HALF = 0.7071067811865476


def _gelu(x):
    return 0.5 * x * (1.0 + lax.erf(x * _SQRT_HALF))


def _pre_body(x_ref, pg, pb, gg, gb, h_ref, z0_ref, zb_ref):
    x = x_ref[...]
    h = _gelu(_ln(x, pg[...], pb[...]))
    h_ref[...] = h
    z0 = jax.nn.relu(_ln(h[:, DG:], gg[...], gb[...]))
    z0_ref[...] = z0
    zb_ref[...] = z0.astype(jnp.bfloat16)


BR = 2000
_GRID = N // BR

def _bs_nd(d):
    return pl.BlockSpec((BR, d), lambda i: (i, 0))

def _bs_w(shape):
    return pl.BlockSpec(shape, lambda i: tuple(0 for _ in shape))

_bs_aggp = pl.BlockSpec((NC, BR, DG), lambda i: (0, i, 0))
_bs_degp = pl.BlockSpec((NC, BR, DW), lambda i: (0, i, 0))


_tc_pre = pl.pallas_call(
    _pre_body,
    grid=(_GRID,),
    in_specs=[_bs_nd(D), _bs_w((1, D)), _bs_w((1, D)),
              _bs_w((1, DG)), _bs_w((1, DG))],
    out_specs=(_bs_nd(D), _bs_nd(DG), _bs_nd(DG)),
    out_shape=(jax.ShapeDtypeStruct((N, D), jnp.float32),
               jax.ShapeDtypeStruct((N, DG), jnp.float32),
               jax.ShapeDtypeStruct((N, DG), jnp.bfloat16)),
)


def _agg_from_partials(aggp_ref, degp_ref):
    agg = aggp_ref[0] + aggp_ref[1]
    deg = degp_ref[0] + degp_ref[1]
    deg = jnp.maximum(deg, 1.0)
    return agg / deg[:, :1]


def _mid_body(h_ref, z_ref, aggp_ref, degp_ref, wl, bl, wr, br, gg, gb,
              y0_ref, z1_ref, zb_ref):
    z = z_ref[...]
    agg = _agg_from_partials(aggp_ref, degp_ref)
    conv = (jnp.dot(agg, wl[...], preferred_element_type=jnp.float32) + bl[...]
            + jnp.dot(z, wr[...], preferred_element_type=jnp.float32) + br[...])
    y0 = h_ref[:, :DG] + conv
    y0_ref[...] = y0
    z1 = jax.nn.relu(_ln(y0, gg[...], gb[...]))
    z1_ref[...] = z1
    zb_ref[...] = z1.astype(jnp.bfloat16)


_tc_mid = pl.pallas_call(
    _mid_body,
    grid=(_GRID,),
    in_specs=[_bs_nd(D), _bs_nd(DG), _bs_aggp, _bs_degp,
              _bs_w((DG, DG)), _bs_w((1, DG)), _bs_w((DG, DG)), _bs_w((1, DG)),
              _bs_w((1, DG)), _bs_w((1, DG))],
    out_specs=(_bs_nd(DG), _bs_nd(DG), _bs_nd(DG)),
    out_shape=(jax.ShapeDtypeStruct((N, DG), jnp.float32),
               jax.ShapeDtypeStruct((N, DG), jnp.float32),
               jax.ShapeDtypeStruct((N, DG), jnp.bfloat16)),
)


def _post_x(h_ref, y0_ref, z1_ref, aggp_ref, degp_ref, wl, bl, wr, br,
            xres_ref, og, ob):
    z1 = z1_ref[...]
    agg = _agg_from_partials(aggp_ref, degp_ref)
    conv = (jnp.dot(agg, wl[...], preferred_element_type=jnp.float32) + bl[...]
            + jnp.dot(z1, wr[...], preferred_element_type=jnp.float32) + br[...])
    y1 = h_ref[:, DG:] + conv
    hcat = jnp.concatenate([y0_ref[...], y1], axis=-1)
    return _ln(hcat + xres_ref[...], og[...], ob[...])


def _postpre_body(h_ref, y0_ref, z1_ref, aggp_ref, degp_ref, wl, bl, wr, br,
                  xres_ref, og, ob, pg, pb, gg, gb,
                  xn_ref, hn_ref, zn_ref, zb_ref):
    xn = _post_x(h_ref, y0_ref, z1_ref, aggp_ref, degp_ref, wl, bl, wr, br,
                 xres_ref, og, ob)
    xn_ref[...] = xn
    hn = _gelu(_ln(xn, pg[...], pb[...]))
    hn_ref[...] = hn
    zn = jax.nn.relu(_ln(hn[:, DG:], gg[...], gb[...]))
    zn_ref[...] = zn
    zb_ref[...] = zn.astype(jnp.bfloat16)


_tc_postpre = pl.pallas_call(
    _postpre_body,
    grid=(_GRID,),
    in_specs=[_bs_nd(D), _bs_nd(DG), _bs_nd(DG), _bs_aggp, _bs_degp,
              _bs_w((DG, DG)), _bs_w((1, DG)), _bs_w((DG, DG)), _bs_w((1, DG)),
              _bs_nd(D), _bs_w((1, D)), _bs_w((1, D)),
              _bs_w((1, D)), _bs_w((1, D)), _bs_w((1, DG)), _bs_w((1, DG))],
    out_specs=(_bs_nd(D), _bs_nd(D), _bs_nd(DG), _bs_nd(DG)),
    out_shape=(jax.ShapeDtypeStruct((N, D), jnp.float32),
               jax.ShapeDtypeStruct((N, D), jnp.float32),
               jax.ShapeDtypeStruct((N, DG), jnp.float32),
               jax.ShapeDtypeStruct((N, DG), jnp.bfloat16)),
)


def _postgelu_body(h_ref, y0_ref, z1_ref, aggp_ref, degp_ref, wl, bl, wr, br,
                   xres_ref, og, ob, out_ref):
    xn = _post_x(h_ref, y0_ref, z1_ref, aggp_ref, degp_ref, wl, bl, wr, br,
                 xres_ref, og, ob)
    out_ref[...] = _gelu(xn)


_tc_postgelu = pl.pallas_call(
    _postgelu_body,
    grid=(_GRID,),
    in_specs=[_bs_nd(D), _bs_nd(DG), _bs_nd(DG), _bs_aggp, _bs_degp,
              _bs_w((DG, DG)), _bs_w((1, DG)), _bs_w((DG, DG)), _bs_w((1, DG)),
              _bs_nd(D), _bs_w((1, D)), _bs_w((1, D))],
    out_specs=_bs_nd(D),
    out_shape=jax.ShapeDtypeStruct((N, D), jnp.float32),
)


def kernel(x, edge_index, edge_weights, edge_attr, pre_ln_g, pre_ln_b,
           grp_ln_g, grp_ln_b, Wl, bl, Wr, br, post_ln_g, post_ln_b,
           et_ln_g, et_ln_b, et_W, et_b, en_g, en_b):
    edge2d = edge_index.reshape(2, E // K, K)
    Wlt = jnp.swapaxes(Wl, -1, -2)[:, :, _PERM, :]
    Wrt = jnp.swapaxes(Wr, -1, -2)
    xres = x
    h, z0, zb = _tc_pre(x, pre_ln_g[0][None], pre_ln_b[0][None],
                        grp_ln_g[0, 0][None], grp_ln_b[0, 0][None])
    aggp, degp = _sc_segsum_deg(edge2d, zb)
    for l in range(L):
        y0, z1, zb = _tc_mid(h, z0, aggp, degp, Wlt[l, 0], bl[l, 0][None],
                             Wrt[l, 0], br[l, 0][None],
                             grp_ln_g[l, 1][None], grp_ln_b[l, 1][None])
        aggp = _sc_segsum(edge2d, zb)
        if l < L - 1:
            xres, h, z0, zb = _tc_postpre(
                h, y0, z1, aggp, degp, Wlt[l, 1], bl[l, 1][None],
                Wrt[l, 1], br[l, 1][None], xres,
                post_ln_g[l][None], post_ln_b[l][None],
                pre_ln_g[l + 1][None], pre_ln_b[l + 1][None],
                grp_ln_g[l + 1, 0][None], grp_ln_b[l + 1, 0][None])
            aggp = _sc_segsum(edge2d, zb)
        else:
            x = _tc_postgelu(h, y0, z1, aggp, degp, Wlt[l, 1], bl[l, 1][None],
                             Wrt[l, 1], br[l, 1][None], xres,
                             post_ln_g[l][None], post_ln_b[l][None])
    return x


# speedup vs baseline: 1.4987x; 1.4987x over previous
import functools

import jax
import jax.numpy as jnp
from jax import lax
from jax.experimental import pallas as pl
from jax.experimental.pallas import tpu as pltpu
from jax.experimental.pallas import tpu_sc as plsc

N, E, D, DG, L = 10000, 320000, 128, 64, 4
NPAD = 10240
NC, NS = 2, 16
NW = NC * NS
EPW = E // NW
K = 125
NCHUNK = EPW // K
NBUF = 5
NROUND = NCHUNK // NBUF
RPT = NPAD // NS
DW = 16

_mesh = plsc.VectorSubcoreMesh(
    core_axis_name="c", subcore_axis_name="s", num_cores=NC, num_subcores=NS)
_sc_params = pltpu.CompilerParams(use_tc_tiling_on_sc=False)


@functools.partial(
    pl.kernel,
    out_type=jax.ShapeDtypeStruct((NC, NPAD, DG), jnp.float32),
    mesh=_mesh,
    compiler_params=_sc_params,
    scratch_types=[
        pltpu.VMEM((NCHUNK, K), jnp.int32),
        pltpu.VMEM((NCHUNK, K), jnp.int32),
        pltpu.VMEM((NBUF, K, DG), jnp.float32),
        pltpu.VMEM((64, DG), jnp.float32),
        pltpu.VMEM_SHARED((NPAD, DG), jnp.float32),
        pltpu.SemaphoreType.DMA((NBUF,)),
        pltpu.SemaphoreType.DMA((NBUF,)),
        pltpu.SemaphoreType.DMA,
        pltpu.SemaphoreType.DMA,
    ],
)
def _sc_segsum(edge_hbm, z_hbm, out_hbm, sidx, didx, rows, zbuf, acc,
               gsem, ssem, isem, zsem):
    c = lax.axis_index("c")
    s = lax.axis_index("s")
    wid = s * NC + c
    zero16 = jnp.zeros((16,), jnp.float32)

    ild0 = pltpu.async_copy(edge_hbm.at[0, pl.ds(wid * NCHUNK, NCHUNK)], sidx,
                            isem)
    ild1 = pltpu.async_copy(edge_hbm.at[1, pl.ds(wid * NCHUNK, NCHUNK)], didx,
                            isem)

    def _zb(i, carry):
        zbuf[i // (DG // 16), pl.ds((i % (DG // 16)) * 16, 16)] = zero16
        return carry

    lax.fori_loop(0, 64 * (DG // 16), _zb, 0)

    def _zc(j, carry):
        pltpu.async_copy(zbuf, acc.at[pl.ds(s * RPT + j * 64, 64)], zsem)
        return carry

    lax.fori_loop(0, RPT // 64, _zc, 0)

    def _zd(j, carry):
        pltpu.make_async_copy(zbuf, acc.at[pl.ds(0, 64)], zsem).wait()
        return carry

    lax.fori_loop(0, RPT // 64, _zd, 0)
    ild0.wait()
    ild1.wait()
    plsc.subcore_barrier()

    for b in range(NBUF):
        pltpu.async_copy(z_hbm.at[sidx.at[b]], rows.at[b], gsem.at[b])

    def _round(g, carry):
        descs = []
        for b in range(NBUF):
            pltpu.make_async_copy(z_hbm.at[pl.ds(0, K)], rows.at[b],
                                  gsem.at[b]).wait()
            descs.append(pltpu.async_copy(rows.at[b], acc.at[didx.at[g * NBUF + b]],
                                          ssem.at[b], add=True))
        for b in range(NBUF):
            descs[b].wait()

            @pl.when(g + 1 < NROUND)
            def _():
                pltpu.async_copy(z_hbm.at[sidx.at[(g + 1) * NBUF + b]],
                                 rows.at[b], gsem.at[b])
        return carry

    lax.fori_loop(0, NROUND, _round, 0)
    plsc.subcore_barrier()
    pltpu.sync_copy(acc.at[pl.ds(s * RPT, RPT)],
                    out_hbm.at[c, pl.ds(s * RPT, RPT)])


@functools.partial(
    pl.kernel,
    out_type=(jax.ShapeDtypeStruct((NC, NPAD, DG), jnp.float32),
              jax.ShapeDtypeStruct((NC, NPAD, DW), jnp.float32)),
    mesh=_mesh,
    compiler_params=_sc_params,
    scratch_types=[
        pltpu.VMEM((NCHUNK, K), jnp.int32),
        pltpu.VMEM((NCHUNK, K), jnp.int32),
        pltpu.VMEM((NBUF, K, DG), jnp.float32),
        pltpu.VMEM((64, DG), jnp.float32),
        pltpu.VMEM((K, DW), jnp.float32),
        pltpu.VMEM((64, DW), jnp.float32),
        pltpu.VMEM_SHARED((NPAD, DG), jnp.float32),
        pltpu.VMEM_SHARED((NPAD, DW), jnp.float32),
        pltpu.SemaphoreType.DMA((NBUF,)),
        pltpu.SemaphoreType.DMA((NBUF,)),
        pltpu.SemaphoreType.DMA,
        pltpu.SemaphoreType.DMA,
        pltpu.SemaphoreType.DMA,
    ],
)
def _sc_segsum_deg(edge_hbm, z_hbm, out_hbm, deg_hbm, sidx, didx, rows,
                   zbuf, ones, zbuf2, acc, dacc, gsem, ssem, isem, zsem, dsem):
    c = lax.axis_index("c")
    s = lax.axis_index("s")
    wid = s * NC + c
    zero16 = jnp.zeros((16,), jnp.float32)
    one16 = jnp.ones((16,), jnp.float32)

    ild0 = pltpu.async_copy(edge_hbm.at[0, pl.ds(wid * NCHUNK, NCHUNK)], sidx,
                            isem)
    ild1 = pltpu.async_copy(edge_hbm.at[1, pl.ds(wid * NCHUNK, NCHUNK)], didx,
                            isem)

    def _zb(i, carry):
        zbuf[i // (DG // 16), pl.ds((i % (DG // 16)) * 16, 16)] = zero16
        return carry

    lax.fori_loop(0, 64 * (DG // 16), _zb, 0)

    def _fill(i, carry):
        ones[i, :] = one16
        zbuf2[i % 64, :] = zero16
        return carry

    lax.fori_loop(0, K, _fill, 0)

    def _zc(j, carry):
        pltpu.async_copy(zbuf, acc.at[pl.ds(s * RPT + j * 64, 64)], zsem)
        pltpu.async_copy(zbuf2, dacc.at[pl.ds(s * RPT + j * 64, 64)], zsem)
        return carry

    lax.fori_loop(0, RPT // 64, _zc, 0)

    def _zd(j, carry):
        pltpu.make_async_copy(zbuf, acc.at[pl.ds(0, 64)], zsem).wait()
        pltpu.make_async_copy(zbuf2, dacc.at[pl.ds(0, 64)], zsem).wait()
        return carry

    lax.fori_loop(0, RPT // 64, _zd, 0)
    ild0.wait()
    ild1.wait()
    plsc.subcore_barrier()

    for b in range(NBUF):
        pltpu.async_copy(z_hbm.at[sidx.at[b]], rows.at[b], gsem.at[b])

    def _round(g, carry):
        descs = []
        for b in range(NBUF):
            i = g * NBUF + b
            pltpu.make_async_copy(z_hbm.at[pl.ds(0, K)], rows.at[b],
                                  gsem.at[b]).wait()
            descs.append(pltpu.async_copy(rows.at[b], acc.at[didx.at[i]],
                                          ssem.at[b], add=True))
            pltpu.async_copy(ones, dacc.at[didx.at[i]], dsem, add=True)
        for b in range(NBUF):
            descs[b].wait()

            @pl.when(g + 1 < NROUND)
            def _():
                pltpu.async_copy(z_hbm.at[sidx.at[(g + 1) * NBUF + b]],
                                 rows.at[b], gsem.at[b])
        return carry

    lax.fori_loop(0, NROUND, _round, 0)

    def _ddrain(i, carry):
        pltpu.make_async_copy(ones, dacc.at[didx.at[0]], dsem).wait()
        return carry

    lax.fori_loop(0, NCHUNK, _ddrain, 0)
    plsc.subcore_barrier()
    pltpu.sync_copy(acc.at[pl.ds(s * RPT, RPT)],
                    out_hbm.at[c, pl.ds(s * RPT, RPT)])
    pltpu.sync_copy(dacc.at[pl.ds(s * RPT, RPT)],
                    deg_hbm.at[c, pl.ds(s * RPT, RPT)])


def _ln(x, g, b, eps=1e-5):
    m = jnp.mean(x, axis=-1, keepdims=True)
    v = jnp.mean((x - m) ** 2, axis=-1, keepdims=True)
    return (x - m) / jnp.sqrt(v + eps) * g + b


_SQRT_HALF = 0.7071067811865476


def _gelu(x):
    return 0.5 * x * (1.0 + lax.erf(x * _SQRT_HALF))


def _pre_body(x_ref, pg, pb, gg, gb, h_ref, z0_ref):
    x = x_ref[...]
    h = _gelu(_ln(x, pg[...], pb[...]))
    h_ref[...] = h
    z0_ref[...] = jax.nn.relu(_ln(h[:, DG:], gg[...], gb[...]))


BR = 2000
_GRID = N // BR

def _bs_nd(d):
    return pl.BlockSpec((BR, d), lambda i: (i, 0))

def _bs_w(shape):
    return pl.BlockSpec(shape, lambda i: tuple(0 for _ in shape))

_bs_aggp = pl.BlockSpec((NC, BR, DG), lambda i: (0, i, 0))
_bs_degp = pl.BlockSpec((NC, BR, DW), lambda i: (0, i, 0))


_tc_pre = pl.pallas_call(
    _pre_body,
    grid=(_GRID,),
    in_specs=[_bs_nd(D), _bs_w((1, D)), _bs_w((1, D)),
              _bs_w((1, DG)), _bs_w((1, DG))],
    out_specs=(_bs_nd(D), _bs_nd(DG)),
    out_shape=(jax.ShapeDtypeStruct((N, D), jnp.float32),
               jax.ShapeDtypeStruct((N, DG), jnp.float32)),
)


def _agg_from_partials(aggp_ref, degp_ref):
    agg = aggp_ref[0] + aggp_ref[1]
    deg = degp_ref[0] + degp_ref[1]
    deg = jnp.maximum(deg, 1.0)
    return agg / deg[:, :1]


def _mid_body(h_ref, z_ref, aggp_ref, degp_ref, wl, bl, wr, br, gg, gb,
              y0_ref, z1_ref):
    z = z_ref[...]
    agg = _agg_from_partials(aggp_ref, degp_ref)
    conv = (jnp.dot(agg, wl[...], preferred_element_type=jnp.float32) + bl[...]
            + jnp.dot(z, wr[...], preferred_element_type=jnp.float32) + br[...])
    y0 = h_ref[:, :DG] + conv
    y0_ref[...] = y0
    z1_ref[...] = jax.nn.relu(_ln(y0, gg[...], gb[...]))


_tc_mid = pl.pallas_call(
    _mid_body,
    grid=(_GRID,),
    in_specs=[_bs_nd(D), _bs_nd(DG), _bs_aggp, _bs_degp,
              _bs_w((DG, DG)), _bs_w((1, DG)), _bs_w((DG, DG)), _bs_w((1, DG)),
              _bs_w((1, DG)), _bs_w((1, DG))],
    out_specs=(_bs_nd(DG), _bs_nd(DG)),
    out_shape=(jax.ShapeDtypeStruct((N, DG), jnp.float32),
               jax.ShapeDtypeStruct((N, DG), jnp.float32)),
)


def _post_x(h_ref, y0_ref, z1_ref, aggp_ref, degp_ref, wl, bl, wr, br,
            xres_ref, og, ob):
    z1 = z1_ref[...]
    agg = _agg_from_partials(aggp_ref, degp_ref)
    conv = (jnp.dot(agg, wl[...], preferred_element_type=jnp.float32) + bl[...]
            + jnp.dot(z1, wr[...], preferred_element_type=jnp.float32) + br[...])
    y1 = h_ref[:, DG:] + conv
    hcat = jnp.concatenate([y0_ref[...], y1], axis=-1)
    return _ln(hcat + xres_ref[...], og[...], ob[...])


def _postpre_body(h_ref, y0_ref, z1_ref, aggp_ref, degp_ref, wl, bl, wr, br,
                  xres_ref, og, ob, pg, pb, gg, gb,
                  xn_ref, hn_ref, zn_ref):
    xn = _post_x(h_ref, y0_ref, z1_ref, aggp_ref, degp_ref, wl, bl, wr, br,
                 xres_ref, og, ob)
    xn_ref[...] = xn
    hn = _gelu(_ln(xn, pg[...], pb[...]))
    hn_ref[...] = hn
    zn_ref[...] = jax.nn.relu(_ln(hn[:, DG:], gg[...], gb[...]))


_tc_postpre = pl.pallas_call(
    _postpre_body,
    grid=(_GRID,),
    in_specs=[_bs_nd(D), _bs_nd(DG), _bs_nd(DG), _bs_aggp, _bs_degp,
              _bs_w((DG, DG)), _bs_w((1, DG)), _bs_w((DG, DG)), _bs_w((1, DG)),
              _bs_nd(D), _bs_w((1, D)), _bs_w((1, D)),
              _bs_w((1, D)), _bs_w((1, D)), _bs_w((1, DG)), _bs_w((1, DG))],
    out_specs=(_bs_nd(D), _bs_nd(D), _bs_nd(DG)),
    out_shape=(jax.ShapeDtypeStruct((N, D), jnp.float32),
               jax.ShapeDtypeStruct((N, D), jnp.float32),
               jax.ShapeDtypeStruct((N, DG), jnp.float32)),
)


def _postgelu_body(h_ref, y0_ref, z1_ref, aggp_ref, degp_ref, wl, bl, wr, br,
                   xres_ref, og, ob, out_ref):
    xn = _post_x(h_ref, y0_ref, z1_ref, aggp_ref, degp_ref, wl, bl, wr, br,
                 xres_ref, og, ob)
    out_ref[...] = _gelu(xn)


_tc_postgelu = pl.pallas_call(
    _postgelu_body,
    grid=(_GRID,),
    in_specs=[_bs_nd(D), _bs_nd(DG), _bs_nd(DG), _bs_aggp, _bs_degp,
              _bs_w((DG, DG)), _bs_w((1, DG)), _bs_w((DG, DG)), _bs_w((1, DG)),
              _bs_nd(D), _bs_w((1, D)), _bs_w((1, D))],
    out_specs=_bs_nd(D),
    out_shape=jax.ShapeDtypeStruct((N, D), jnp.float32),
)


def kernel(x, edge_index, edge_weights, edge_attr, pre_ln_g, pre_ln_b,
           grp_ln_g, grp_ln_b, Wl, bl, Wr, br, post_ln_g, post_ln_b,
           et_ln_g, et_ln_b, et_W, et_b, en_g, en_b):
    edge2d = edge_index.reshape(2, E // K, K)
    Wlt = jnp.swapaxes(Wl, -1, -2)
    Wrt = jnp.swapaxes(Wr, -1, -2)
    xres = x
    h, z0 = _tc_pre(x, pre_ln_g[0][None], pre_ln_b[0][None],
                    grp_ln_g[0, 0][None], grp_ln_b[0, 0][None])
    aggp, degp = _sc_segsum_deg(edge2d, z0)
    for l in range(L):
        y0, z1 = _tc_mid(h, z0, aggp, degp, Wlt[l, 0], bl[l, 0][None],
                         Wrt[l, 0], br[l, 0][None],
                         grp_ln_g[l, 1][None], grp_ln_b[l, 1][None])
        aggp = _sc_segsum(edge2d, z1)
        if l < L - 1:
            xres, h, z0 = _tc_postpre(
                h, y0, z1, aggp, degp, Wlt[l, 1], bl[l, 1][None],
                Wrt[l, 1], br[l, 1][None], xres,
                post_ln_g[l][None], post_ln_b[l][None],
                pre_ln_g[l + 1][None], pre_ln_b[l + 1][None],
                grp_ln_g[l + 1, 0][None], grp_ln_b[l + 1, 0][None])
            aggp = _sc_segsum(edge2d, z0)
        else:
            x = _tc_postgelu(h, y0, z1, aggp, degp, Wlt[l, 1], bl[l, 1][None],
                             Wrt[l, 1], br[l, 1][None], xres,
                             post_ln_g[l][None], post_ln_b[l][None])
    return x
